# baseline (speedup 1.0000x reference)
import jax
import jax.numpy as jnp
from jax import lax
from jax.experimental import pallas as pl
from jax.experimental.pallas import tpu as pltpu
from jax.experimental.pallas import tpu_sc as plsc

VOCAB = 1000000
EMBED = 64
SEQ = 200
BATCH = 4096

NC = 2
NS = 16
NW = NC * NS
L = 16
NEG = EMBED // L

ST = SEQ // 8
BT = BATCH // 128
ET = EMBED // 8
NQ = 4
NI = 128 * NQ


def _embed_kernel(x4_hbm, tab_hbm, pos_hbm, out_hbm,
                  idx_w, pos_v, gidx0, gidx1, rows0, rows1, tile0, tile1,
                  sem_g0, sem_g1, sem_w0, sem_w1):
    wid = lax.axis_index("s") * NC + lax.axis_index("c")

    pltpu.sync_copy(x4_hbm.at[:, wid], idx_w)
    pltpu.sync_copy(pos_hbm, pos_v)

    gidx = (gidx0, gidx1)
    rows = (rows0, rows1)
    tile = (tile0, tile1)
    sem_g = (sem_g0, sem_g1)
    sem_w = (sem_w0, sem_w1)

    iot = jax.lax.iota(jnp.int32, L)
    zerov = jnp.bitwise_and(iot, 0)
    perm = [jnp.right_shift(iot, 2) + 4 * q for q in range(NQ)]
    quarter = jnp.bitwise_and(iot, 3)
    dnums = lax.GatherDimensionNumbers(
        offset_dims=(), collapsed_slice_dims=(0,), start_index_map=(0,))

    def lane_perm(v, p):
        return lax.gather(v, p[:, None], dimension_numbers=dnums,
                          slice_sizes=(1,),
                          mode=lax.GatherScatterMode.PROMISE_IN_BOUNDS)

    def gather_copies(b):
        return [
            pltpu.make_async_copy(
                tab_hbm.at[gidx[b].at[pl.ds(j * 128, 128)]],
                rows[b].at[pl.ds(j * 128, 128)],
                sem_g[b],
            )
            for j in range(NI // 128)
        ]

    def fire_gather(b, s):
        ts = s // 8
        ss = s % 8

        def g_body(g, carry):
            tokv = idx_w[ts, ss, pl.ds(g * L, L)]
            for q in range(NQ):
                ex = lane_perm(tokv, perm[q])
                gidx[b][pl.ds(g * NQ * L + q * L, L)] = (
                    jnp.left_shift(ex, 2) + quarter)
            return carry

        lax.fori_loop(0, 128 // L, g_body, 0)
        for cp in gather_copies(b):
            cp.start()

    def wb(b, s):
        return [
            pltpu.make_async_copy(
                tile[b].at[pl.ds(eb * 1024, 1024)],
                out_hbm.at[s, eb, wid],
                sem_w[b],
            )
            for eb in range(ET)
        ]

    iota4 = iot * NQ
    colc = [zerov + c for c in range(L)]

    def pos_add(b, s):
        pp = [pos_v[s, pl.ds(p * L, L)] for p in range(NQ)]

        def t_body(t, carry):
            for p in range(NQ):
                r = t * NQ + p
                rows[b][r, :] = rows[b][r, :] + pp[p]
            return carry

        lax.fori_loop(0, 128, t_body, 0)

    def transpose_rows(b, s):
        def l_body(l, carry):
            base = iota4 + l * (L * NQ)
            rb = [base + q for q in range(NQ)]
            for e in range(EMBED):
                vals = plsc.load_gather(rows[b], [rb[e // L], colc[e % L]])
                off = (e // 8) * 1024 + (e % 8) * 128 + l * L
                tile[b][pl.ds(off, L)] = vals
            return carry

        lax.fori_loop(0, 128 // L, l_body, 0)

    fire_gather(0, 0)
    for c in gather_copies(0):
        c.wait()
    fire_gather(1, 1)
    pos_add(0, 0)
    transpose_rows(0, 0)
    for c in wb(0, 0):
        c.start()

    def s_body(s2, carry):
        for b in range(2):
            s = 1 + 2 * s2 + b
            bb = (1 + b) % 2
            for c in gather_copies(bb):
                c.wait()
            pos_add(bb, s)
            transpose_rows(bb, s)
            for c in wb(1 - bb, s - 1):
                c.wait()
            fire_gather(1 - bb, s + 1)
            for c in wb(bb, s):
                c.start()
        return carry

    lax.fori_loop(0, (SEQ - 2) // 2, s_body, 0)

    for c in gather_copies(1):
        c.wait()
    pos_add(1, SEQ - 1)
    transpose_rows(1, SEQ - 1)
    for c in wb(0, SEQ - 2):
        c.wait()
    for c in wb(1, SEQ - 1):
        c.start()
    for c in wb(1, SEQ - 1):
        c.wait()


@jax.jit
def _embed(x4, tab4, position_table):
    mesh = plsc.VectorSubcoreMesh(core_axis_name="c", subcore_axis_name="s")
    return pl.kernel(
        _embed_kernel,
        mesh=mesh,
        out_type=jax.ShapeDtypeStruct((SEQ, ET, BT, 1024), jnp.float32),
        scratch_types=[
            pltpu.VMEM((ST, 8, 128), jnp.int32),
            pltpu.VMEM((SEQ, EMBED), jnp.float32),
            pltpu.VMEM((NI,), jnp.int32),
            pltpu.VMEM((NI,), jnp.int32),
            pltpu.VMEM((NI, L), jnp.float32),
            pltpu.VMEM((NI, L), jnp.float32),
            pltpu.VMEM((ET * 1024,), jnp.float32),
            pltpu.VMEM((ET * 1024,), jnp.float32),
            pltpu.SemaphoreType.DMA,
            pltpu.SemaphoreType.DMA,
            pltpu.SemaphoreType.DMA,
            pltpu.SemaphoreType.DMA,
        ],
        compiler_params=pltpu.CompilerParams(
            use_tc_tiling_on_sc=False, needs_layout_passes=False),
    )(x4, tab4, position_table)


def kernel(x, token_table, position_table):
    x4 = x.reshape(BT, 128, ST, 8).transpose(2, 0, 3, 1)
    tab4 = token_table.reshape(VOCAB * NQ, L)
    out5 = _embed(x4, tab4, position_table).reshape(SEQ, ET, BT, 8, 128)
    return out5.transpose(2, 4, 0, 1, 3).reshape(BATCH, SEQ, EMBED)


# speedup vs baseline: 1.7691x; 1.7691x over previous
import jax
import jax.numpy as jnp
from jax import lax
from jax.experimental import pallas as pl
from jax.experimental.pallas import tpu as pltpu
from jax.experimental.pallas import tpu_sc as plsc

VOCAB = 1000000
EMBED = 64
SEQ = 200
BATCH = 4096

NC = 2
NS = 16
NW = NC * NS

TOTAL = BATCH * SEQ
ROWS_PER_W = TOTAL // NW
SEQS_PER_CHUNK = 4
CHUNK = SEQS_PER_CHUNK * SEQ
NCHUNK = ROWS_PER_W // CHUNK
NDMA = 8
ROWS_PER_DMA = CHUNK // NDMA


def _embed_kernel(idx_hbm, tok_hbm, pos_hbm, out_hbm,
                  idx0, idx1, rows0, rows1, pos_v,
                  sem_g0, sem_g1, sem_w0, sem_w1):
    wid = lax.axis_index("s") * NC + lax.axis_index("c")
    row0 = wid * ROWS_PER_W
    irow0 = wid * (ROWS_PER_W // ROWS_PER_DMA)
    idx_v = (idx0, idx1)
    rows_v = (rows0, rows1)
    sem_g = (sem_g0, sem_g1)
    sem_w = (sem_w0, sem_w1)

    pltpu.sync_copy(pos_hbm, pos_v)

    def gather_copies(b):
        return [
            pltpu.make_async_copy(
                tok_hbm.at[idx_v[b].at[j]],
                rows_v[b].at[pl.ds(j * ROWS_PER_DMA, ROWS_PER_DMA)],
                sem_g[b],
            )
            for j in range(NDMA)
        ]

    def copy_idx(b, c):
        pltpu.sync_copy(idx_hbm.at[pl.ds(irow0 + c * NDMA, NDMA)], idx_v[b])

    def wb_copy(b, c):
        return pltpu.make_async_copy(
            rows_v[b], out_hbm.at[pl.ds(row0 + c * CHUNK, CHUNK)], sem_w[b])

    def pos_add(b):
        def body(p, carry):
            for g in range(EMBED // 16):
                sl = pl.ds(g * 16, 16)
                pv = pos_v[p, sl]
                for s in range(SEQS_PER_CHUNK):
                    r = s * SEQ + p
                    rows_v[b][r, sl] = rows_v[b][r, sl] + pv
            return carry

        lax.fori_loop(0, SEQ, body, 0)

    copy_idx(0, 0)
    for cp in gather_copies(0):
        cp.start()

    def service(b, c, prep_next):
        if prep_next:
            copy_idx(1 - b, c + 1)
            for cp in gather_copies(1 - b):
                cp.start()
        for cp in gather_copies(b):
            cp.wait()
        pos_add(b)
        wb_copy(b, c).start()

    service(0, 0, True)

    def pair_body(g2, carry):
        for b in range(2):
            c = 1 + 2 * g2 + b
            bb = (1 + b) % 2
            wb_copy(1 - bb, c - 1).wait()
            service(bb, c, True)
        return carry

    lax.fori_loop(0, (NCHUNK - 2) // 2, pair_body, 0)

    wb_copy(0, NCHUNK - 2).wait()
    service(1, NCHUNK - 1, False)
    wb_copy(1, NCHUNK - 1).wait()


@jax.jit
def _embed(idx2d, token_table, position_table):
    mesh = plsc.VectorSubcoreMesh(core_axis_name="c", subcore_axis_name="s")
    return pl.kernel(
        _embed_kernel,
        mesh=mesh,
        out_type=jax.ShapeDtypeStruct((TOTAL, EMBED), jnp.float32),
        scratch_types=[
            pltpu.VMEM((NDMA, ROWS_PER_DMA), jnp.int32),
            pltpu.VMEM((NDMA, ROWS_PER_DMA), jnp.int32),
            pltpu.VMEM((CHUNK, EMBED), jnp.float32),
            pltpu.VMEM((CHUNK, EMBED), jnp.float32),
            pltpu.VMEM((SEQ, EMBED), jnp.float32),
            pltpu.SemaphoreType.DMA,
            pltpu.SemaphoreType.DMA,
            pltpu.SemaphoreType.DMA,
            pltpu.SemaphoreType.DMA,
        ],
        compiler_params=pltpu.CompilerParams(use_tc_tiling_on_sc=False),
    )(idx2d, token_table, position_table)


def kernel(x, token_table, position_table):
    idx2d = x.reshape(TOTAL // ROWS_PER_DMA, ROWS_PER_DMA)
    out = _embed(idx2d, token_table, position_table)
    return out.reshape(BATCH, SEQ, EMBED)
